# baseline (speedup 1.0000x reference)
import functools

import jax
import jax.numpy as jnp
from jax import lax
from jax.experimental import pallas as pl
from jax.experimental.pallas import tpu as pltpu
from jax.experimental.pallas import tpu_sc as plsc

VOCAB = 100000
DIM = 128
B = 4096
L = 50
N = B * L


@functools.lru_cache(maxsize=None)
def _build_gather(nbuf=8, chunk=80, look=7, nchunk_c0=82, nchunk_c1=78):
    info = plsc.get_sparse_core_info()
    nc, ns = info.num_cores, info.num_subcores
    assert nc == 2 and ns == 16
    len0, len1 = nchunk_c0 * chunk, nchunk_c1 * chunk
    pair = len0 + len1
    assert pair * ns == N
    len_max = max(len0, len1)
    assert chunk % 8 == 0 and len0 % 8 == 0 and look < nbuf
    assert min(nchunk_c0, nchunk_c1) >= nbuf >= look

    mesh = plsc.VectorSubcoreMesh(core_axis_name="c", subcore_axis_name="s")

    @functools.partial(
        pl.kernel,
        out_type=jax.ShapeDtypeStruct((N, DIM), jnp.float32),
        mesh=mesh,
        scratch_types=[
            pltpu.VMEM((len_max,), jnp.int32),
            pltpu.VMEM((nbuf, chunk, DIM), jnp.float32),
        ]
        + [pltpu.SemaphoreType.DMA] * (2 * nbuf),
    )
    def gather_kernel(table_hbm, idx_hbm, out_hbm, idx_v, rows_v, *sems):
        gsem, osem = sems[:nbuf], sems[nbuf:]
        c = lax.axis_index("c")
        s = lax.axis_index("s")
        base = s * pair + c * len0
        nchunk_w = jnp.where(c == 0, nchunk_c0, nchunk_c1)
        pltpu.sync_copy(idx_hbm.at[pl.ds(base, len_max)], idx_v)

        def start_gather(i, b):
            pltpu.async_copy(
                table_hbm.at[idx_v.at[pl.ds(i * chunk, chunk)]],
                rows_v.at[b],
                gsem[b],
            )

        for cc in range(look):
            start_gather(cc, cc)

        ngroups = (nchunk_w + nbuf - 1) // nbuf

        @pl.loop(0, ngroups)
        def _(g0):
            g = g0 * nbuf
            for b in range(nbuf):
                i = g + b

                @pl.when(i < nchunk_w)
                def _():
                    pltpu.make_async_copy(
                        table_hbm.at[idx_v.at[pl.ds(0, chunk)]],
                        rows_v.at[b],
                        gsem[b],
                    ).wait()
                    pltpu.async_copy(
                        rows_v.at[b],
                        out_hbm.at[pl.ds(base + i * chunk, chunk)],
                        osem[b],
                    )
                    j = i + look
                    bj = (b + look) % nbuf

                    @pl.when(jnp.logical_and(j >= nbuf, j < nchunk_w))
                    def _():
                        pltpu.make_async_copy(
                            rows_v.at[bj],
                            out_hbm.at[pl.ds(base, chunk)],
                            osem[bj],
                        ).wait()

                    @pl.when(j < nchunk_w)
                    def _():
                        start_gather(j, bj)

        for b in range(nbuf):
            pltpu.make_async_copy(
                rows_v.at[b], out_hbm.at[pl.ds(base, chunk)], osem[b]
            ).wait()

    return gather_kernel


def kernel(state, input_token, table):
    idx_t = input_token.astype(jnp.int32).T.reshape(N)
    rows = _build_gather()(table, idx_t)
    hidden = rows.reshape(L, B, DIM).transpose(1, 0, 2)
    return (state, hidden)


# speedup vs baseline: 1.0134x; 1.0134x over previous
import functools

import jax
import jax.numpy as jnp
from jax import lax
from jax.experimental import pallas as pl
from jax.experimental.pallas import tpu as pltpu
from jax.experimental.pallas import tpu_sc as plsc

VOCAB = 100000
DIM = 128
B = 4096
L = 50
N = B * L


@functools.lru_cache(maxsize=None)
def _build_gather(nbuf=8, chunk=80, look=7, nchunk_c0=78, nchunk_c1=82):
    info = plsc.get_sparse_core_info()
    nc, ns = info.num_cores, info.num_subcores
    assert nc == 2 and ns == 16
    len0, len1 = nchunk_c0 * chunk, nchunk_c1 * chunk
    pair = len0 + len1
    assert pair * ns == N
    len_max = max(len0, len1)
    assert chunk % 8 == 0 and len0 % 8 == 0 and look < nbuf
    assert min(nchunk_c0, nchunk_c1) >= nbuf >= look

    mesh = plsc.VectorSubcoreMesh(core_axis_name="c", subcore_axis_name="s")

    @functools.partial(
        pl.kernel,
        out_type=jax.ShapeDtypeStruct((N, DIM), jnp.float32),
        mesh=mesh,
        scratch_types=[
            pltpu.VMEM((len_max,), jnp.int32),
            pltpu.VMEM((nbuf, chunk, DIM), jnp.float32),
        ]
        + [pltpu.SemaphoreType.DMA] * (2 * nbuf),
    )
    def gather_kernel(table_hbm, idx_hbm, out_hbm, idx_v, rows_v, *sems):
        gsem, osem = sems[:nbuf], sems[nbuf:]
        c = lax.axis_index("c")
        s = lax.axis_index("s")
        base = s * pair + c * len0
        nchunk_w = jnp.where(c == 0, nchunk_c0, nchunk_c1)
        pltpu.sync_copy(idx_hbm.at[pl.ds(base, len_max)], idx_v)

        def start_gather(i, b):
            pltpu.async_copy(
                table_hbm.at[idx_v.at[pl.ds(i * chunk, chunk)]],
                rows_v.at[b],
                gsem[b],
            )

        for cc in range(look):
            start_gather(cc, cc)

        ngroups = (nchunk_w + nbuf - 1) // nbuf

        @pl.loop(0, ngroups)
        def _(g0):
            g = g0 * nbuf
            for b in range(nbuf):
                i = g + b

                @pl.when(i < nchunk_w)
                def _():
                    pltpu.make_async_copy(
                        table_hbm.at[idx_v.at[pl.ds(0, chunk)]],
                        rows_v.at[b],
                        gsem[b],
                    ).wait()
                    pltpu.async_copy(
                        rows_v.at[b],
                        out_hbm.at[pl.ds(base + i * chunk, chunk)],
                        osem[b],
                    )
                    j = i + look
                    bj = (b + look) % nbuf

                    @pl.when(jnp.logical_and(j >= nbuf, j < nchunk_w))
                    def _():
                        pltpu.make_async_copy(
                            rows_v.at[bj],
                            out_hbm.at[pl.ds(base, chunk)],
                            osem[bj],
                        ).wait()

                    @pl.when(j < nchunk_w)
                    def _():
                        start_gather(j, bj)

        for b in range(nbuf):
            pltpu.make_async_copy(
                rows_v.at[b], out_hbm.at[pl.ds(base, chunk)], osem[b]
            ).wait()

    return gather_kernel


def kernel(state, input_token, table):
    idx_t = input_token.astype(jnp.int32).T.reshape(N)
    rows = _build_gather()(table, idx_t)
    hidden = rows.reshape(L, B, DIM).transpose(1, 0, 2)
    return (state, hidden)
